# Initial kernel scaffold; baseline (speedup 1.0000x reference)
import jax
import jax.numpy as jnp
from jax.experimental import pallas as pl


def kernel(x, edge_index, W, b):
    raise NotImplementedError("write your pallas kernel here")


# speedup vs baseline: 12.3645x; 12.3645x over previous
import jax
import jax.numpy as jnp
from jax import lax
from jax.experimental import pallas as pl
from jax.experimental.pallas import tpu as pltpu
from jax.experimental.pallas import tpu_sc as plsc

N_REAL = 10000
D = 128
D_IN = 256
NC, NS = 2, 16
ROWS_PER_SUB = 640
N_PAD = NS * ROWS_PER_SUB
W_EDGES = 128
N_WIN = 40
E_PAD = NC * NS * N_WIN * W_EDGES
BLK = 512

_MESH = plsc.VectorSubcoreMesh(core_axis_name="c", subcore_axis_name="s")


def _hop_body(z_hbm, src_hbm, dst_hbm, out_hbm, srcw_v, dstw_v, rows_v,
              acc_sh, sem):
    c = lax.axis_index("c")
    s = lax.axis_index("s")
    slab = pl.ds(s * ROWS_PER_SUB, ROWS_PER_SUB)

    @pl.when(c == 0)
    def _():
        pltpu.sync_copy(z_hbm.at[slab], acc_sh.at[slab])

    @pl.when(c == 1)
    def _():
        @pl.loop(0, W_EDGES)
        def _(i):
            @pl.loop(0, D, step=16)
            def _(j):
                rows_v[i, pl.ds(j, 16)] = jnp.zeros((16,), jnp.float32)

        @pl.loop(0, ROWS_PER_SUB, step=W_EDGES)
        def _(r):
            pltpu.sync_copy(
                rows_v, acc_sh.at[pl.ds(s * ROWS_PER_SUB + r, W_EDGES)])

    plsc.subcore_barrier()

    base = (c * NS + s) * N_WIN

    @pl.loop(0, N_WIN)
    def _(j):
        pltpu.sync_copy(src_hbm.at[base + j], srcw_v)
        pltpu.sync_copy(dst_hbm.at[base + j], dstw_v)
        pltpu.async_copy(z_hbm.at[srcw_v], rows_v, sem).wait()
        pltpu.sync_copy(rows_v, acc_sh.at[dstw_v], add=True)

    plsc.subcore_barrier()
    pltpu.sync_copy(acc_sh.at[slab], out_hbm.at[c].at[slab])


@jax.jit
def _sc_hop(z, src2d, dst2d):
    return pl.kernel(
        _hop_body,
        out_type=jax.ShapeDtypeStruct((NC, N_PAD, D), jnp.float32),
        mesh=_MESH,
        scratch_types=[
            pltpu.VMEM((W_EDGES,), jnp.int32),
            pltpu.VMEM((W_EDGES,), jnp.int32),
            pltpu.VMEM((W_EDGES, D), jnp.float32),
            pltpu.VMEM_SHARED((N_PAD, D), jnp.float32),
            pltpu.SemaphoreType.DMA,
        ],
    )(z, src2d, dst2d)


def _mm_body(x_ref, w_ref, o_ref):
    o_ref[...] = lax.dot_general(
        x_ref[...], w_ref[...], (((1,), (1,)), ((), ())),
        preferred_element_type=jnp.float32,
        precision=lax.Precision.HIGHEST,
    )


@jax.jit
def _tc_mm(x_pad, w):
    return pl.pallas_call(
        _mm_body,
        grid=(N_PAD // BLK,),
        in_specs=[
            pl.BlockSpec((BLK, D_IN), lambda i: (i, 0)),
            pl.BlockSpec((D, D_IN), lambda i: (0, 0)),
        ],
        out_specs=pl.BlockSpec((BLK, D), lambda i: (i, 0)),
        out_shape=jax.ShapeDtypeStruct((N_PAD, D), jnp.float32),
    )(x_pad, w)


def _z_body(y_ref, pd_ref, z_ref, degb_ref):
    degb = pd_ref[0] + pd_ref[1]
    degb_ref[...] = degb
    z_ref[...] = y_ref[...] * lax.rsqrt(degb)


@jax.jit
def _tc_z(y, pd):
    return pl.pallas_call(
        _z_body,
        grid=(N_PAD // BLK,),
        in_specs=[
            pl.BlockSpec((BLK, D), lambda i: (i, 0)),
            pl.BlockSpec((NC, BLK, D), lambda i: (0, i, 0)),
        ],
        out_specs=[
            pl.BlockSpec((BLK, D), lambda i: (i, 0)),
            pl.BlockSpec((BLK, D), lambda i: (i, 0)),
        ],
        out_shape=[
            jax.ShapeDtypeStruct((N_PAD, D), jnp.float32),
            jax.ShapeDtypeStruct((N_PAD, D), jnp.float32),
        ],
    )(y, pd)


def _mid_body(p_ref, degb_ref, v_ref):
    v_ref[...] = (p_ref[0] + p_ref[1]) / degb_ref[...]


@jax.jit
def _tc_mid(p, degb):
    return pl.pallas_call(
        _mid_body,
        grid=(N_PAD // BLK,),
        in_specs=[
            pl.BlockSpec((NC, BLK, D), lambda i: (0, i, 0)),
            pl.BlockSpec((BLK, D), lambda i: (i, 0)),
        ],
        out_specs=pl.BlockSpec((BLK, D), lambda i: (i, 0)),
        out_shape=jax.ShapeDtypeStruct((N_PAD, D), jnp.float32),
    )(p, degb)


def _out_body(q_ref, degb_ref, b_ref, o_ref):
    o = (q_ref[0] + q_ref[1]) * lax.rsqrt(degb_ref[...]) + b_ref[...]
    m = jnp.max(o, axis=1, keepdims=True)
    e = jnp.exp(o - m)
    ssum = jnp.sum(e, axis=1, keepdims=True)
    o_ref[...] = o - m - jnp.log(ssum)


@jax.jit
def _tc_out(q, degb, b2d):
    return pl.pallas_call(
        _out_body,
        grid=(N_PAD // BLK,),
        in_specs=[
            pl.BlockSpec((NC, BLK, D), lambda i: (0, i, 0)),
            pl.BlockSpec((BLK, D), lambda i: (i, 0)),
            pl.BlockSpec((1, D), lambda i: (0, 0)),
        ],
        out_specs=pl.BlockSpec((BLK, D), lambda i: (i, 0)),
        out_shape=jax.ShapeDtypeStruct((N_PAD, D), jnp.float32),
    )(q, degb, b2d)


def kernel(x, edge_index, W, b):
    src = edge_index[0].astype(jnp.int32)
    dst = edge_index[1].astype(jnp.int32)
    n_extra = E_PAD - src.shape[0]
    sink = (jnp.arange(n_extra, dtype=jnp.int32) % 8) + N_REAL
    src2d = jnp.concatenate([src, sink]).reshape(E_PAD // W_EDGES, W_EDGES)
    dst2d = jnp.concatenate([dst, sink]).reshape(E_PAD // W_EDGES, W_EDGES)
    x_pad = jnp.pad(x, ((0, N_PAD - N_REAL), (0, 0)))
    ones = jnp.ones((N_PAD, D), jnp.float32)

    pd = _sc_hop(ones, src2d, dst2d)
    y = _tc_mm(x_pad, W)
    z, degb = _tc_z(y, pd)
    p1 = _sc_hop(z, src2d, dst2d)
    v = _tc_mid(p1, degb)
    p2 = _sc_hop(v, src2d, dst2d)
    out = _tc_out(p2, degb, b.reshape(1, D))
    return out[:N_REAL]


# speedup vs baseline: 19.2438x; 1.5564x over previous
import jax
import jax.numpy as jnp
from jax import lax
from jax.experimental import pallas as pl
from jax.experimental.pallas import tpu as pltpu
from jax.experimental.pallas import tpu_sc as plsc

N_REAL = 10000
D = 128
D_IN = 256
NC, NS = 2, 16
ROWS_PER_SUB = 640
N_PAD = NS * ROWS_PER_SUB
W_EDGES = 128
N_WIN = 40
E_PAD = NC * NS * N_WIN * W_EDGES
BLK = 512

_MESH = plsc.VectorSubcoreMesh(core_axis_name="c", subcore_axis_name="s")


def _hop_body(z_hbm, src_hbm, dst_hbm, out_hbm,
              srcw0, srcw1, dstw0, dstw1, rows0, rows1,
              acc_sh, sg0, sg1, ss0, ss1):
    c = lax.axis_index("c")
    s = lax.axis_index("s")
    slab = pl.ds(s * ROWS_PER_SUB, ROWS_PER_SUB)

    @pl.when(c == 0)
    def _():
        pltpu.sync_copy(z_hbm.at[slab], acc_sh.at[slab])

    @pl.when(c == 1)
    def _():
        @pl.loop(0, W_EDGES)
        def _(i):
            @pl.loop(0, D, step=16)
            def _(j):
                rows0[i, pl.ds(j, 16)] = jnp.zeros((16,), jnp.float32)

        @pl.loop(0, ROWS_PER_SUB, step=W_EDGES)
        def _(r):
            pltpu.sync_copy(
                rows0, acc_sh.at[pl.ds(s * ROWS_PER_SUB + r, W_EDGES)])

    plsc.subcore_barrier()

    base = (c * NS + s) * N_WIN

    pltpu.sync_copy(src_hbm.at[base], srcw0)
    pltpu.sync_copy(dst_hbm.at[base], dstw0)
    pltpu.async_copy(z_hbm.at[srcw0], rows0, sg0)
    pltpu.sync_copy(src_hbm.at[base + 1], srcw1)
    pltpu.sync_copy(dst_hbm.at[base + 1], dstw1)
    pltpu.async_copy(z_hbm.at[srcw1], rows1, sg1)

    @pl.loop(0, N_WIN // 2)
    def _(i):
        a = base + 2 * i
        pltpu.make_async_copy(z_hbm.at[srcw0], rows0, sg0).wait()
        pltpu.async_copy(rows0, acc_sh.at[dstw0], ss0, add=True)
        pltpu.make_async_copy(z_hbm.at[srcw1], rows1, sg1).wait()
        pltpu.async_copy(rows1, acc_sh.at[dstw1], ss1, add=True)

        @pl.when(i < N_WIN // 2 - 1)
        def _():
            pltpu.make_async_copy(rows0, acc_sh.at[dstw0], ss0).wait()
            pltpu.sync_copy(src_hbm.at[a + 2], srcw0)
            pltpu.sync_copy(dst_hbm.at[a + 2], dstw0)
            pltpu.async_copy(z_hbm.at[srcw0], rows0, sg0)
            pltpu.make_async_copy(rows1, acc_sh.at[dstw1], ss1).wait()
            pltpu.sync_copy(src_hbm.at[a + 3], srcw1)
            pltpu.sync_copy(dst_hbm.at[a + 3], dstw1)
            pltpu.async_copy(z_hbm.at[srcw1], rows1, sg1)

    pltpu.make_async_copy(rows0, acc_sh.at[dstw0], ss0).wait()
    pltpu.make_async_copy(rows1, acc_sh.at[dstw1], ss1).wait()

    plsc.subcore_barrier()
    pltpu.sync_copy(acc_sh.at[slab], out_hbm.at[c].at[slab])


@jax.jit
def _sc_hop(z, src2d, dst2d):
    return pl.kernel(
        _hop_body,
        out_type=jax.ShapeDtypeStruct((NC, N_PAD, D), jnp.float32),
        mesh=_MESH,
        scratch_types=[
            pltpu.VMEM((W_EDGES,), jnp.int32),
            pltpu.VMEM((W_EDGES,), jnp.int32),
            pltpu.VMEM((W_EDGES,), jnp.int32),
            pltpu.VMEM((W_EDGES,), jnp.int32),
            pltpu.VMEM((W_EDGES, D), jnp.float32),
            pltpu.VMEM((W_EDGES, D), jnp.float32),
            pltpu.VMEM_SHARED((N_PAD, D), jnp.float32),
            pltpu.SemaphoreType.DMA,
            pltpu.SemaphoreType.DMA,
            pltpu.SemaphoreType.DMA,
            pltpu.SemaphoreType.DMA,
        ],
    )(z, src2d, dst2d)


def _deg_body(dst_hbm, out_hbm, dstw0, dstw1, ones_v, zero_v,
              acc_sh, ss0, ss1):
    c = lax.axis_index("c")
    s = lax.axis_index("s")
    slab = pl.ds(s * ROWS_PER_SUB, ROWS_PER_SUB)

    @pl.loop(0, W_EDGES)
    def _(i):
        @pl.loop(0, D, step=16)
        def _(j):
            ones_v[i, pl.ds(j, 16)] = jnp.full((16,), 1.0, jnp.float32)

    @pl.when(c == 1)
    def _():
        @pl.loop(0, W_EDGES)
        def _(i):
            @pl.loop(0, D, step=16)
            def _(j):
                zero_v[i, pl.ds(j, 16)] = jnp.zeros((16,), jnp.float32)

    @pl.loop(0, ROWS_PER_SUB, step=W_EDGES)
    def _(r):
        tgt = acc_sh.at[pl.ds(s * ROWS_PER_SUB + r, W_EDGES)]

        @pl.when(c == 0)
        def _():
            pltpu.sync_copy(ones_v, tgt)

        @pl.when(c == 1)
        def _():
            pltpu.sync_copy(zero_v, tgt)

    plsc.subcore_barrier()

    base = (c * NS + s) * N_WIN

    pltpu.sync_copy(dst_hbm.at[base], dstw0)
    pltpu.async_copy(ones_v, acc_sh.at[dstw0], ss0, add=True)
    pltpu.sync_copy(dst_hbm.at[base + 1], dstw1)
    pltpu.async_copy(ones_v, acc_sh.at[dstw1], ss1, add=True)

    @pl.loop(0, N_WIN // 2)
    def _(i):
        a = base + 2 * i

        @pl.when(i < N_WIN // 2 - 1)
        def _():
            pltpu.make_async_copy(ones_v, acc_sh.at[dstw0], ss0).wait()
            pltpu.sync_copy(dst_hbm.at[a + 2], dstw0)
            pltpu.async_copy(ones_v, acc_sh.at[dstw0], ss0, add=True)
            pltpu.make_async_copy(ones_v, acc_sh.at[dstw1], ss1).wait()
            pltpu.sync_copy(dst_hbm.at[a + 3], dstw1)
            pltpu.async_copy(ones_v, acc_sh.at[dstw1], ss1, add=True)

    pltpu.make_async_copy(ones_v, acc_sh.at[dstw0], ss0).wait()
    pltpu.make_async_copy(ones_v, acc_sh.at[dstw1], ss1).wait()

    plsc.subcore_barrier()
    pltpu.sync_copy(acc_sh.at[slab], out_hbm.at[c].at[slab])


@jax.jit
def _sc_deg(dst2d):
    return pl.kernel(
        _deg_body,
        out_type=jax.ShapeDtypeStruct((NC, N_PAD, D), jnp.float32),
        mesh=_MESH,
        scratch_types=[
            pltpu.VMEM((W_EDGES,), jnp.int32),
            pltpu.VMEM((W_EDGES,), jnp.int32),
            pltpu.VMEM((W_EDGES, D), jnp.float32),
            pltpu.VMEM((W_EDGES, D), jnp.float32),
            pltpu.VMEM_SHARED((N_PAD, D), jnp.float32),
            pltpu.SemaphoreType.DMA,
            pltpu.SemaphoreType.DMA,
        ],
    )(dst2d)


def _mm_body(x_ref, w_ref, o_ref):
    o_ref[...] = lax.dot_general(
        x_ref[...], w_ref[...], (((1,), (1,)), ((), ())),
        preferred_element_type=jnp.float32,
        precision=lax.Precision.HIGHEST,
    )


@jax.jit
def _tc_mm(x_pad, w):
    return pl.pallas_call(
        _mm_body,
        grid=(N_PAD // BLK,),
        in_specs=[
            pl.BlockSpec((BLK, D_IN), lambda i: (i, 0)),
            pl.BlockSpec((D, D_IN), lambda i: (0, 0)),
        ],
        out_specs=pl.BlockSpec((BLK, D), lambda i: (i, 0)),
        out_shape=jax.ShapeDtypeStruct((N_PAD, D), jnp.float32),
    )(x_pad, w)


def _z_body(y_ref, pd_ref, z_ref, degb_ref):
    degb = pd_ref[0] + pd_ref[1]
    degb_ref[...] = degb
    z_ref[...] = y_ref[...] * lax.rsqrt(degb)


@jax.jit
def _tc_z(y, pd):
    return pl.pallas_call(
        _z_body,
        grid=(N_PAD // BLK,),
        in_specs=[
            pl.BlockSpec((BLK, D), lambda i: (i, 0)),
            pl.BlockSpec((NC, BLK, D), lambda i: (0, i, 0)),
        ],
        out_specs=[
            pl.BlockSpec((BLK, D), lambda i: (i, 0)),
            pl.BlockSpec((BLK, D), lambda i: (i, 0)),
        ],
        out_shape=[
            jax.ShapeDtypeStruct((N_PAD, D), jnp.float32),
            jax.ShapeDtypeStruct((N_PAD, D), jnp.float32),
        ],
    )(y, pd)


def _mid_body(p_ref, degb_ref, v_ref):
    v_ref[...] = (p_ref[0] + p_ref[1]) / degb_ref[...]


@jax.jit
def _tc_mid(p, degb):
    return pl.pallas_call(
        _mid_body,
        grid=(N_PAD // BLK,),
        in_specs=[
            pl.BlockSpec((NC, BLK, D), lambda i: (0, i, 0)),
            pl.BlockSpec((BLK, D), lambda i: (i, 0)),
        ],
        out_specs=pl.BlockSpec((BLK, D), lambda i: (i, 0)),
        out_shape=jax.ShapeDtypeStruct((N_PAD, D), jnp.float32),
    )(p, degb)


def _out_body(q_ref, degb_ref, b_ref, o_ref):
    o = (q_ref[0] + q_ref[1]) * lax.rsqrt(degb_ref[...]) + b_ref[...]
    m = jnp.max(o, axis=1, keepdims=True)
    e = jnp.exp(o - m)
    ssum = jnp.sum(e, axis=1, keepdims=True)
    o_ref[...] = o - m - jnp.log(ssum)


@jax.jit
def _tc_out(q, degb, b2d):
    return pl.pallas_call(
        _out_body,
        grid=(N_PAD // BLK,),
        in_specs=[
            pl.BlockSpec((NC, BLK, D), lambda i: (0, i, 0)),
            pl.BlockSpec((BLK, D), lambda i: (i, 0)),
            pl.BlockSpec((1, D), lambda i: (0, 0)),
        ],
        out_specs=pl.BlockSpec((BLK, D), lambda i: (i, 0)),
        out_shape=jax.ShapeDtypeStruct((N_PAD, D), jnp.float32),
    )(q, degb, b2d)


def kernel(x, edge_index, W, b):
    src = edge_index[0].astype(jnp.int32)
    dst = edge_index[1].astype(jnp.int32)
    n_extra = E_PAD - src.shape[0]
    sink = (jnp.arange(n_extra, dtype=jnp.int32) % 8) + N_REAL
    src2d = jnp.concatenate([src, sink]).reshape(E_PAD // W_EDGES, W_EDGES)
    dst2d = jnp.concatenate([dst, sink]).reshape(E_PAD // W_EDGES, W_EDGES)
    x_pad = jnp.pad(x, ((0, N_PAD - N_REAL), (0, 0)))

    pd = _sc_deg(dst2d)
    y = _tc_mm(x_pad, W)
    z, degb = _tc_z(y, pd)
    p1 = _sc_hop(z, src2d, dst2d)
    v = _tc_mid(p1, degb)
    p2 = _sc_hop(v, src2d, dst2d)
    out = _tc_out(p2, degb, b.reshape(1, D))
    return out[:N_REAL]
